# speedup vs baseline: 14.4431x; 14.4431x over previous
import functools
import math

import jax
import jax.numpy as jnp
from jax.experimental import pallas as pl
from jax.experimental.pallas import tpu as pltpu

_T = 2048
_C = 768
_H = 12
_HD = _C // _H
_RB = 256
_N_ITER = 30


def _qkv_body(x_ref, w_ref, b_ref, o_ref):
    o_ref[...] = jax.lax.dot_general(
        x_ref[...], w_ref[...], (((1,), (1,)), ((), ())),
        preferred_element_type=jnp.float32) + b_ref[...]


def _attn_body(ratio_ref, q_ref, k_ref, v_ref, o_ref):
    h = pl.program_id(0)
    tb = pl.program_id(1)
    q = q_ref[0]
    k = k_ref[0]
    scale = 1.0 / math.sqrt(_HD)
    att = jax.lax.dot_general(
        q, k, (((1,), (1,)), ((), ())),
        preferred_element_type=jnp.float32) * scale

    rows = tb * _RB + jax.lax.broadcasted_iota(jnp.int32, (_RB, 1), 0)
    cols = jax.lax.broadcasted_iota(jnp.int32, (_RB, _T), 1)
    valid = cols <= rows

    big = jnp.float32(3e38)
    att_m = jnp.where(valid, att, -big)
    hi = jnp.max(att_m, axis=1, keepdims=True)
    lo = jnp.min(jnp.where(valid, att, big), axis=1, keepdims=True)

    r = ratio_ref[h]
    sig = 1.0 / (1.0 + jnp.exp(-r))
    tlen = (rows + 1).astype(jnp.float32)
    kt = jnp.maximum(1, jnp.floor(tlen * sig).astype(jnp.int32))
    ktf = kt.astype(jnp.float32)

    def bisect(_, carry):
        lo_c, hi_c = carry
        mid = (lo_c + hi_c) * 0.5
        cnt = jnp.sum((att_m >= mid).astype(jnp.float32), axis=1,
                      keepdims=True)
        ge = cnt >= ktf
        return jnp.where(ge, mid, lo_c), jnp.where(ge, hi_c, mid)

    lo, hi = jax.lax.fori_loop(0, _N_ITER, bisect, (lo, hi))

    keep = valid & (att >= lo)
    s = jnp.where(keep, att, 0.0)
    m = jnp.maximum(jnp.max(att_m, axis=1, keepdims=True), 0.0)
    p = jnp.exp(s - m)
    w = p / jnp.sum(p, axis=1, keepdims=True)
    o_ref[0] = jax.lax.dot_general(
        w, v_ref[0], (((1,), (0,)), ((), ())),
        preferred_element_type=jnp.float32)


def _proj_body(y_ref, w_ref, b_ref, o_ref):
    o_ref[...] = jax.lax.dot_general(
        y_ref[...], w_ref[...], (((1,), (1,)), ((), ())),
        preferred_element_type=jnp.float32) + b_ref[...]


@jax.jit
def kernel(x, W_attn, b_attn, W_proj, b_proj, sparsity_ratios):
    B, T, C = x.shape
    H = sparsity_ratios.shape[0]
    hd = C // H
    x2 = x.reshape(T, C)

    qkv = pl.pallas_call(
        _qkv_body,
        grid=(T // _RB,),
        in_specs=[
            pl.BlockSpec((_RB, C), lambda i: (i, 0)),
            pl.BlockSpec((3 * C, C), lambda i: (0, 0)),
            pl.BlockSpec((1, 3 * C), lambda i: (0, 0)),
        ],
        out_specs=pl.BlockSpec((_RB, 3 * C), lambda i: (i, 0)),
        out_shape=jax.ShapeDtypeStruct((T, 3 * C), jnp.float32),
    )(x2, W_attn, b_attn.reshape(1, 3 * C))

    q = qkv[:, :C].reshape(T, H, hd).transpose(1, 0, 2)
    k = qkv[:, C:2 * C].reshape(T, H, hd).transpose(1, 0, 2)
    v = qkv[:, 2 * C:].reshape(T, H, hd).transpose(1, 0, 2)

    grid_spec = pltpu.PrefetchScalarGridSpec(
        num_scalar_prefetch=1,
        grid=(H, T // _RB),
        in_specs=[
            pl.BlockSpec((1, _RB, hd), lambda h, t, *_: (h, t, 0)),
            pl.BlockSpec((1, T, hd), lambda h, t, *_: (h, 0, 0)),
            pl.BlockSpec((1, T, hd), lambda h, t, *_: (h, 0, 0)),
        ],
        out_specs=pl.BlockSpec((1, _RB, hd), lambda h, t, *_: (h, t, 0)),
    )
    y = pl.pallas_call(
        _attn_body,
        grid_spec=grid_spec,
        out_shape=jax.ShapeDtypeStruct((H, T, hd), jnp.float32),
        compiler_params=pltpu.CompilerParams(
            dimension_semantics=("arbitrary", "arbitrary")),
    )(sparsity_ratios, q, k, v)

    y2 = y.transpose(1, 0, 2).reshape(T, C)
    out = pl.pallas_call(
        _proj_body,
        grid=(T // _RB,),
        in_specs=[
            pl.BlockSpec((_RB, C), lambda i: (i, 0)),
            pl.BlockSpec((C, C), lambda i: (0, 0)),
            pl.BlockSpec((1, C), lambda i: (0, 0)),
        ],
        out_specs=pl.BlockSpec((_RB, C), lambda i: (i, 0)),
        out_shape=jax.ShapeDtypeStruct((T, C), jnp.float32),
    )(y2, W_proj, b_proj.reshape(1, C))
    return out.reshape(B, T, C)
